# baseline (speedup 1.0000x reference)
import functools

import jax
import jax.numpy as jnp
from jax import lax
from jax.experimental import pallas as pl
from jax.experimental.pallas import tpu as pltpu
from jax.experimental.pallas import tpu_sc as plsc

NN = 10000
NP = 10240
EE = 320000
D_IN = 128
DH = 128
DO = 256
LL = 6
GG = 64

NTILES = 16
EB = 32
NB = 640
CHUNK = 16
NCH = NB // CHUNK
NBUF = 4
EPT = NB * EB
EPAD = EPT * NTILES
RPT = NP // NTILES

RB = 1024
F32 = jnp.float32


def _sc_agg_body(m_lo, m_hi, src_t, dst_t, zrows, agg_lo, agg_hi,
                 src_v, dst_v, buf0, buf1, buf2, buf3, acc,
                 sem0, sem1, sem2, sem3):
    c = lax.axis_index("c")
    s = lax.axis_index("s")
    r0 = s * RPT
    bufs = (buf0, buf1, buf2, buf3)
    sems = (sem0, sem1, sem2, sem3)

    def run(m_ref, out_ref):
        pltpu.sync_copy(zrows, acc.at[pl.ds(r0, RPT)])
        plsc.subcore_barrier()

        def chunk_step(k, carry):
            base = s * NB + k * CHUNK
            pltpu.sync_copy(src_t.at[pl.ds(base, CHUNK)], src_v)
            pltpu.sync_copy(dst_t.at[pl.ds(base, CHUNK)], dst_v)
            cps = [None] * NBUF
            for j in range(NBUF - 1):
                cps[j] = pltpu.async_copy(
                    m_ref.at[src_v.at[j]], bufs[j], sems[j])
            for j in range(CHUNK):
                p = j % NBUF
                jn = j + NBUF - 1
                if jn < CHUNK:
                    pn = jn % NBUF
                    cps[pn] = pltpu.async_copy(
                        m_ref.at[src_v.at[jn]], bufs[pn], sems[pn])
                cps[p].wait()
            return carry

        lax.fori_loop(0, NCH, chunk_step, 0)
        plsc.subcore_barrier()
        pltpu.sync_copy(acc.at[pl.ds(r0, RPT)], out_ref.at[pl.ds(r0, RPT)])

    @pl.when(c == 0)
    def _():
        run(m_lo, agg_lo)

    @pl.when(c == 1)
    def _():
        run(m_hi, agg_hi)


_sc_agg = functools.partial(
    pl.kernel,
    _sc_agg_body,
    out_type=[jax.ShapeDtypeStruct((NP, DH), F32),
              jax.ShapeDtypeStruct((NP, DH), F32)],
    mesh=plsc.VectorSubcoreMesh(core_axis_name="c", subcore_axis_name="s"),
    scratch_types=[
        pltpu.VMEM((CHUNK, EB), jnp.int32),
        pltpu.VMEM((CHUNK, EB), jnp.int32),
        pltpu.VMEM((EB, DO), F32),
        pltpu.VMEM((EB, DO), F32),
        pltpu.VMEM((EB, DO), F32),
        pltpu.VMEM((EB, DO), F32),
        pltpu.VMEM_SHARED((NP, DH), F32),
        pltpu.SemaphoreType.DMA,
        pltpu.SemaphoreType.DMA,
        pltpu.SemaphoreType.DMA,
        pltpu.SemaphoreType.DMA,
    ],
)()


def _mm0_body(x_ref, w_ref, mlo_ref, mhi_ref):
    m = jnp.dot(x_ref[...], w_ref[...], preferred_element_type=F32)
    mlo_ref[...] = m[:, :DH]
    mhi_ref[...] = m[:, DH:]


def _mm0(xp, w_top):
    return pl.pallas_call(
        _mm0_body,
        grid=(NP // RB,),
        in_specs=[
            pl.BlockSpec((RB, D_IN), lambda i: (i, 0)),
            pl.BlockSpec((D_IN, DO), lambda i: (0, 0)),
        ],
        out_specs=[
            pl.BlockSpec((RB, DH), lambda i: (i, 0)),
            pl.BlockSpec((RB, DH), lambda i: (i, 0)),
        ],
        out_shape=[jax.ShapeDtypeStruct((NP, DH), F32),
                   jax.ShapeDtypeStruct((NP, DH), F32)],
    )(xp, w_top)


def _gru_core(alo_ref, ahi_ref, h_ref, wil_ref, wih_ref, whh_ref,
              bih_ref, bhh_ref):
    h = h_ref[...]
    gi = (jnp.dot(alo_ref[...], wil_ref[...], preferred_element_type=F32)
          + jnp.dot(ahi_ref[...], wih_ref[...], preferred_element_type=F32)
          + bih_ref[...])
    gh = jnp.dot(h, whh_ref[...], preferred_element_type=F32) + bhh_ref[...]
    r = jax.nn.sigmoid(gi[:, :DO] + gh[:, :DO])
    z = jax.nn.sigmoid(gi[:, DO:2 * DO] + gh[:, DO:2 * DO])
    nn_ = jnp.tanh(gi[:, 2 * DO:] + r * gh[:, 2 * DO:])
    return (1.0 - z) * nn_ + z * h


def _gru_mid_body(alo_ref, ahi_ref, h_ref, wil_ref, wih_ref, whh_ref,
                  bih_ref, bhh_ref, wn_ref, h_out, mlo_ref, mhi_ref):
    hn = _gru_core(alo_ref, ahi_ref, h_ref, wil_ref, wih_ref, whh_ref,
                   bih_ref, bhh_ref)
    h_out[...] = hn
    m = jnp.dot(hn, wn_ref[...], preferred_element_type=F32)
    mlo_ref[...] = m[:, :DH]
    mhi_ref[...] = m[:, DH:]


def _gru_last_body(alo_ref, ahi_ref, h_ref, wil_ref, wih_ref, whh_ref,
                   bih_ref, bhh_ref, h_out):
    h_out[...] = _gru_core(alo_ref, ahi_ref, h_ref, wil_ref, wih_ref,
                           whh_ref, bih_ref, bhh_ref)


_GRU_IN_SPECS = [
    pl.BlockSpec((RB, DH), lambda i: (i, 0)),
    pl.BlockSpec((RB, DH), lambda i: (i, 0)),
    pl.BlockSpec((RB, DO), lambda i: (i, 0)),
    pl.BlockSpec((DH, 3 * DO), lambda i: (0, 0)),
    pl.BlockSpec((DH, 3 * DO), lambda i: (0, 0)),
    pl.BlockSpec((DO, 3 * DO), lambda i: (0, 0)),
    pl.BlockSpec((1, 3 * DO), lambda i: (0, 0)),
    pl.BlockSpec((1, 3 * DO), lambda i: (0, 0)),
]


def _gru_mid(alo, ahi, h, wil, wih, whh, bih, bhh, wn):
    return pl.pallas_call(
        _gru_mid_body,
        grid=(NP // RB,),
        in_specs=_GRU_IN_SPECS + [pl.BlockSpec((DO, DO), lambda i: (0, 0))],
        out_specs=[
            pl.BlockSpec((RB, DO), lambda i: (i, 0)),
            pl.BlockSpec((RB, DH), lambda i: (i, 0)),
            pl.BlockSpec((RB, DH), lambda i: (i, 0)),
        ],
        out_shape=[jax.ShapeDtypeStruct((NP, DO), F32),
                   jax.ShapeDtypeStruct((NP, DH), F32),
                   jax.ShapeDtypeStruct((NP, DH), F32)],
    )(alo, ahi, h, wil, wih, whh, bih, bhh, wn)


def _gru_last(alo, ahi, h, wil, wih, whh, bih, bhh):
    return pl.pallas_call(
        _gru_last_body,
        grid=(NP // RB,),
        in_specs=_GRU_IN_SPECS,
        out_specs=pl.BlockSpec((RB, DO), lambda i: (i, 0)),
        out_shape=jax.ShapeDtypeStruct((NP, DO), F32),
    )(alo, ahi, h, wil, wih, whh, bih, bhh)


def _pool_mlp_body(h_ref, x_ref, b_ref, w1h_ref, w1x_ref, b1_ref,
                   w2_ref, b2_ref, w3_ref, b3_ref, out_ref):
    ids = b_ref[...]
    onehot = (ids == lax.broadcasted_iota(jnp.int32, (GG, NP), 0)).astype(F32)
    counts = jnp.sum(onehot, axis=1, keepdims=True)
    inv = 1.0 / jnp.clip(counts, 1.0, None)
    ph = jnp.dot(onehot, h_ref[...], preferred_element_type=F32) * inv
    px = jnp.dot(onehot, x_ref[...], preferred_element_type=F32) * inv
    h1 = jnp.maximum(
        jnp.dot(ph, w1h_ref[...], preferred_element_type=F32)
        + jnp.dot(px, w1x_ref[...], preferred_element_type=F32)
        + b1_ref[...], 0.0)
    h2 = jnp.maximum(
        jnp.dot(h1, w2_ref[...], preferred_element_type=F32) + b2_ref[...],
        0.0)
    logits = jnp.sum(h2 * w3_ref[...], axis=1, keepdims=True) + b3_ref[...]
    out_ref[...] = jax.nn.sigmoid(logits)


def _pool_mlp(h, xp, batch2, w1h, w1x, b1, w2t, b2, w3r, b3):
    return pl.pallas_call(
        _pool_mlp_body,
        out_shape=jax.ShapeDtypeStruct((GG, 1), F32),
    )(h, xp, batch2, w1h, w1x, b1, w2t, b2, w3r, b3)


def kernel(x, edge_index, batch, W_ggnn, w_ih, w_hh, b_ih, b_hh,
           W1, b1, W2, b2, W3, b3):
    xp = jnp.pad(x, ((0, NP - NN), (0, 0)))
    h = jnp.pad(x, ((0, NP - NN), (0, DO - D_IN)))

    src = edge_index[0]
    dst = edge_index[1]
    padn = EPAD - EE
    src_p = jnp.concatenate([src, jnp.zeros((padn,), jnp.int32)])
    dst_pad = NN + (jnp.arange(padn, dtype=jnp.int32) % (NP - NN))
    dst_p = jnp.concatenate([dst, dst_pad])
    src_t = src_p.reshape(NTILES * NB, EB)
    dst_t = dst_p.reshape(NTILES * NB, EB)
    zrows = jnp.zeros((RPT, DH), F32)

    wil = w_ih.T[:DH]
    wih = w_ih.T[DH:]
    whh = w_hh.T
    bih2 = b_ih.reshape(1, 3 * DO)
    bhh2 = b_hh.reshape(1, 3 * DO)

    batch2 = jnp.pad(batch, (0, NP - NN), constant_values=GG).reshape(1, NP)
    w1h = W1.T[:DO]
    w1x = W1.T[DO:]
    b1r = b1.reshape(1, 256)
    w2t = W2.T
    b2r = b2.reshape(1, 128)
    w3r = W3.reshape(1, 128)
    b3r = b3.reshape(1, 1)

    mlo, mhi = _mm0(xp, W_ggnn[0][:D_IN])

    for i in range(LL):
        m256 = jnp.concatenate([mlo, mhi], axis=1)
        alo, ahi = _sc_agg(m256, m256, src_t, dst_t, zrows)
        if i == LL - 1:
            h = _gru_last(alo, ahi, h, wil, wih, whh, bih2, bhh2)
        else:
            h, mlo, mhi = _gru_mid(alo, ahi, h, wil, wih, whh, bih2, bhh2,
                                   W_ggnn[i + 1])

    return _pool_mlp(h, xp, batch2, w1h, w1x, b1r, w2t, b2r, w3r, b3r)


# speedup vs baseline: 1.5685x; 1.5685x over previous
import functools

import jax
import jax.numpy as jnp
from jax import lax
from jax.experimental import pallas as pl
from jax.experimental.pallas import tpu as pltpu
from jax.experimental.pallas import tpu_sc as plsc

NN = 10000
NP = 10240
EE = 320000
D_IN = 128
DH = 128
DO = 256
LL = 6
GG = 64

NTILES = 16
EB = 64
NB = 320
CHUNK = 16
NCH = NB // CHUNK
NBUF = 5
EPT = NB * EB
EPAD = EPT * NTILES
RPT = NP // NTILES

RB = 1024
F32 = jnp.float32


def _sc_agg_body(m_lo, m_hi, src_t, dst_t, zrows, agg_lo, agg_hi,
                 src_v, dst_v, buf0, buf1, buf2, buf3, buf4, acc,
                 sem0, sem1, sem2, sem3, sem4):
    c = lax.axis_index("c")
    s = lax.axis_index("s")
    r0 = s * RPT
    bufs = (buf0, buf1, buf2, buf3, buf4)
    sems = (sem0, sem1, sem2, sem3, sem4)

    def run(m_ref, out_ref):
        pltpu.sync_copy(zrows, acc.at[pl.ds(r0, RPT)])
        plsc.subcore_barrier()

        def chunk_step(k, carry):
            base = s * NB + k * CHUNK
            pltpu.sync_copy(src_t.at[pl.ds(base, CHUNK)], src_v)
            pltpu.sync_copy(dst_t.at[pl.ds(base, CHUNK)], dst_v)
            cps = [None] * NBUF
            for j in range(NBUF - 1):
                cps[j] = pltpu.async_copy(
                    m_ref.at[src_v.at[j]], bufs[j], sems[j])
            for j in range(CHUNK):
                p = j % NBUF
                jn = j + NBUF - 1
                if jn < CHUNK:
                    pn = jn % NBUF
                    cps[pn] = pltpu.async_copy(
                        m_ref.at[src_v.at[jn]], bufs[pn], sems[pn])
                cps[p].wait()
                pltpu.sync_copy(bufs[p], acc.at[dst_v.at[j]], add=True)
            return carry

        lax.fori_loop(0, NCH, chunk_step, 0)
        plsc.subcore_barrier()
        pltpu.sync_copy(acc.at[pl.ds(r0, RPT)], out_ref.at[pl.ds(r0, RPT)])

    @pl.when(c == 0)
    def _():
        run(m_lo, agg_lo)

    @pl.when(c == 1)
    def _():
        run(m_hi, agg_hi)


_sc_agg = functools.partial(
    pl.kernel,
    _sc_agg_body,
    out_type=[jax.ShapeDtypeStruct((NP, DH), F32),
              jax.ShapeDtypeStruct((NP, DH), F32)],
    mesh=plsc.VectorSubcoreMesh(core_axis_name="c", subcore_axis_name="s"),
    scratch_types=[
        pltpu.VMEM((CHUNK, EB), jnp.int32),
        pltpu.VMEM((CHUNK, EB), jnp.int32),
        pltpu.VMEM((EB, DH), F32),
        pltpu.VMEM((EB, DH), F32),
        pltpu.VMEM((EB, DH), F32),
        pltpu.VMEM((EB, DH), F32),
        pltpu.VMEM((EB, DH), F32),
        pltpu.VMEM_SHARED((NP, DH), F32),
        pltpu.SemaphoreType.DMA,
        pltpu.SemaphoreType.DMA,
        pltpu.SemaphoreType.DMA,
        pltpu.SemaphoreType.DMA,
        pltpu.SemaphoreType.DMA,
    ],
)()


def _mm0_body(x_ref, w_ref, mlo_ref, mhi_ref):
    m = jnp.dot(x_ref[...], w_ref[...], preferred_element_type=F32)
    mlo_ref[...] = m[:, :DH]
    mhi_ref[...] = m[:, DH:]


def _mm0(xp, w_top):
    return pl.pallas_call(
        _mm0_body,
        grid=(NP // RB,),
        in_specs=[
            pl.BlockSpec((RB, D_IN), lambda i: (i, 0)),
            pl.BlockSpec((D_IN, DO), lambda i: (0, 0)),
        ],
        out_specs=[
            pl.BlockSpec((RB, DH), lambda i: (i, 0)),
            pl.BlockSpec((RB, DH), lambda i: (i, 0)),
        ],
        out_shape=[jax.ShapeDtypeStruct((NP, DH), F32),
                   jax.ShapeDtypeStruct((NP, DH), F32)],
    )(xp, w_top)


def _gru_core(alo_ref, ahi_ref, h_ref, wil_ref, wih_ref, whh_ref,
              bih_ref, bhh_ref):
    h = h_ref[...]
    gi = (jnp.dot(alo_ref[...], wil_ref[...], preferred_element_type=F32)
          + jnp.dot(ahi_ref[...], wih_ref[...], preferred_element_type=F32)
          + bih_ref[...])
    gh = jnp.dot(h, whh_ref[...], preferred_element_type=F32) + bhh_ref[...]
    r = jax.nn.sigmoid(gi[:, :DO] + gh[:, :DO])
    z = jax.nn.sigmoid(gi[:, DO:2 * DO] + gh[:, DO:2 * DO])
    nn_ = jnp.tanh(gi[:, 2 * DO:] + r * gh[:, 2 * DO:])
    return (1.0 - z) * nn_ + z * h


def _gru_mid_body(alo_ref, ahi_ref, h_ref, wil_ref, wih_ref, whh_ref,
                  bih_ref, bhh_ref, wn_ref, h_out, mlo_ref, mhi_ref):
    hn = _gru_core(alo_ref, ahi_ref, h_ref, wil_ref, wih_ref, whh_ref,
                   bih_ref, bhh_ref)
    h_out[...] = hn
    m = jnp.dot(hn, wn_ref[...], preferred_element_type=F32)
    mlo_ref[...] = m[:, :DH]
    mhi_ref[...] = m[:, DH:]


def _gru_last_body(alo_ref, ahi_ref, h_ref, wil_ref, wih_ref, whh_ref,
                   bih_ref, bhh_ref, h_out):
    h_out[...] = _gru_core(alo_ref, ahi_ref, h_ref, wil_ref, wih_ref,
                           whh_ref, bih_ref, bhh_ref)


_GRU_IN_SPECS = [
    pl.BlockSpec((RB, DH), lambda i: (i, 0)),
    pl.BlockSpec((RB, DH), lambda i: (i, 0)),
    pl.BlockSpec((RB, DO), lambda i: (i, 0)),
    pl.BlockSpec((DH, 3 * DO), lambda i: (0, 0)),
    pl.BlockSpec((DH, 3 * DO), lambda i: (0, 0)),
    pl.BlockSpec((DO, 3 * DO), lambda i: (0, 0)),
    pl.BlockSpec((1, 3 * DO), lambda i: (0, 0)),
    pl.BlockSpec((1, 3 * DO), lambda i: (0, 0)),
]


def _gru_mid(alo, ahi, h, wil, wih, whh, bih, bhh, wn):
    return pl.pallas_call(
        _gru_mid_body,
        grid=(NP // RB,),
        in_specs=_GRU_IN_SPECS + [pl.BlockSpec((DO, DO), lambda i: (0, 0))],
        out_specs=[
            pl.BlockSpec((RB, DO), lambda i: (i, 0)),
            pl.BlockSpec((RB, DH), lambda i: (i, 0)),
            pl.BlockSpec((RB, DH), lambda i: (i, 0)),
        ],
        out_shape=[jax.ShapeDtypeStruct((NP, DO), F32),
                   jax.ShapeDtypeStruct((NP, DH), F32),
                   jax.ShapeDtypeStruct((NP, DH), F32)],
    )(alo, ahi, h, wil, wih, whh, bih, bhh, wn)


def _gru_last(alo, ahi, h, wil, wih, whh, bih, bhh):
    return pl.pallas_call(
        _gru_last_body,
        grid=(NP // RB,),
        in_specs=_GRU_IN_SPECS,
        out_specs=pl.BlockSpec((RB, DO), lambda i: (i, 0)),
        out_shape=jax.ShapeDtypeStruct((NP, DO), F32),
    )(alo, ahi, h, wil, wih, whh, bih, bhh)


def _pool_mlp_body(h_ref, x_ref, b_ref, w1h_ref, w1x_ref, b1_ref,
                   w2_ref, b2_ref, w3_ref, b3_ref, out_ref):
    ids = b_ref[...]
    onehot = (ids == lax.broadcasted_iota(jnp.int32, (GG, NP), 0)).astype(F32)
    counts = jnp.sum(onehot, axis=1, keepdims=True)
    inv = 1.0 / jnp.clip(counts, 1.0, None)
    ph = jnp.dot(onehot, h_ref[...], preferred_element_type=F32) * inv
    px = jnp.dot(onehot, x_ref[...], preferred_element_type=F32) * inv
    h1 = jnp.maximum(
        jnp.dot(ph, w1h_ref[...], preferred_element_type=F32)
        + jnp.dot(px, w1x_ref[...], preferred_element_type=F32)
        + b1_ref[...], 0.0)
    h2 = jnp.maximum(
        jnp.dot(h1, w2_ref[...], preferred_element_type=F32) + b2_ref[...],
        0.0)
    logits = jnp.sum(h2 * w3_ref[...], axis=1, keepdims=True) + b3_ref[...]
    out_ref[...] = jax.nn.sigmoid(logits)


def _pool_mlp(h, xp, batch2, w1h, w1x, b1, w2t, b2, w3r, b3):
    return pl.pallas_call(
        _pool_mlp_body,
        out_shape=jax.ShapeDtypeStruct((GG, 1), F32),
    )(h, xp, batch2, w1h, w1x, b1, w2t, b2, w3r, b3)


def kernel(x, edge_index, batch, W_ggnn, w_ih, w_hh, b_ih, b_hh,
           W1, b1, W2, b2, W3, b3):
    xp = jnp.pad(x, ((0, NP - NN), (0, 0)))
    h = jnp.pad(x, ((0, NP - NN), (0, DO - D_IN)))

    src = edge_index[0]
    dst = edge_index[1]
    padn = EPAD - EE
    src_p = jnp.concatenate([src, jnp.zeros((padn,), jnp.int32)])
    dst_pad = NN + (jnp.arange(padn, dtype=jnp.int32) % (NP - NN))
    dst_p = jnp.concatenate([dst, dst_pad])
    src_t = src_p.reshape(NTILES * NB, EB)
    dst_t = dst_p.reshape(NTILES * NB, EB)
    zrows = jnp.zeros((RPT, DH), F32)

    wil = w_ih.T[:DH]
    wih = w_ih.T[DH:]
    whh = w_hh.T
    bih2 = b_ih.reshape(1, 3 * DO)
    bhh2 = b_hh.reshape(1, 3 * DO)

    batch2 = jnp.pad(batch, (0, NP - NN), constant_values=GG).reshape(1, NP)
    w1h = W1.T[:DO]
    w1x = W1.T[DO:]
    b1r = b1.reshape(1, 256)
    w2t = W2.T
    b2r = b2.reshape(1, 128)
    w3r = W3.reshape(1, 128)
    b3r = b3.reshape(1, 1)

    mlo, mhi = _mm0(xp, W_ggnn[0][:D_IN])

    for i in range(LL):
        alo, ahi = _sc_agg(mlo, mhi, src_t, dst_t, zrows)
        if i == LL - 1:
            h = _gru_last(alo, ahi, h, wil, wih, whh, bih2, bhh2)
        else:
            h, mlo, mhi = _gru_mid(alo, ahi, h, wil, wih, whh, bih2, bhh2,
                                   W_ggnn[i + 1])

    return _pool_mlp(h, xp, batch2, w1h, w1x, b1r, w2t, b2r, w3r, b3r)


# speedup vs baseline: 1.5711x; 1.0016x over previous
import functools

import jax
import jax.numpy as jnp
from jax import lax
from jax.experimental import pallas as pl
from jax.experimental.pallas import tpu as pltpu
from jax.experimental.pallas import tpu_sc as plsc

NN = 10000
NP = 10240
EE = 320000
D_IN = 128
DH = 128
DO = 256
LL = 6
GG = 64

NTILES = 16
EB = 64
NB = 320
CHUNK = 16
NCH = NB // CHUNK
NBUF = 5
EPT = NB * EB
EPAD = EPT * NTILES
RPT = NP // NTILES

RB = 1024
F32 = jnp.float32


def _sc_agg_body(m_lo, m_hi, src_t, dst_t, zrows, agg_lo, agg_hi,
                 src_v, dst_v, buf0, buf1, buf2, buf3, buf4, acc,
                 sem0, sem1, sem2, sem3, sem4,
                 ssem0, ssem1, ssem2, ssem3, ssem4):
    c = lax.axis_index("c")
    s = lax.axis_index("s")
    r0 = s * RPT
    bufs = (buf0, buf1, buf2, buf3, buf4)
    sems = (sem0, sem1, sem2, sem3, sem4)
    ssems = (ssem0, ssem1, ssem2, ssem3, ssem4)

    def run(m_ref, out_ref):
        pltpu.sync_copy(zrows, acc.at[pl.ds(r0, RPT)])
        plsc.subcore_barrier()

        def chunk_step(k, carry):
            base = s * NB + k * CHUNK
            pltpu.sync_copy(src_t.at[pl.ds(base, CHUNK)], src_v)
            pltpu.sync_copy(dst_t.at[pl.ds(base, CHUNK)], dst_v)
            cps = [None] * NBUF
            scps = [None] * NBUF
            for j in range(NBUF - 1):
                cps[j] = pltpu.async_copy(
                    m_ref.at[src_v.at[j]], bufs[j], sems[j])
            for j in range(CHUNK):
                p = j % NBUF
                jn = j + NBUF - 1
                if jn < CHUNK:
                    pn = jn % NBUF
                    if scps[pn] is not None:
                        scps[pn].wait()
                    cps[pn] = pltpu.async_copy(
                        m_ref.at[src_v.at[jn]], bufs[pn], sems[pn])
                cps[p].wait()
                scps[p] = pltpu.async_copy(
                    bufs[p], acc.at[dst_v.at[j]], ssems[p], add=True)
            for p in range(NBUF):
                if scps[p] is not None:
                    scps[p].wait()
            return carry

        lax.fori_loop(0, NCH, chunk_step, 0)
        plsc.subcore_barrier()
        pltpu.sync_copy(acc.at[pl.ds(r0, RPT)], out_ref.at[pl.ds(r0, RPT)])

    @pl.when(c == 0)
    def _():
        run(m_lo, agg_lo)

    @pl.when(c == 1)
    def _():
        run(m_hi, agg_hi)


_sc_agg = functools.partial(
    pl.kernel,
    _sc_agg_body,
    out_type=[jax.ShapeDtypeStruct((NP, DH), F32),
              jax.ShapeDtypeStruct((NP, DH), F32)],
    mesh=plsc.VectorSubcoreMesh(core_axis_name="c", subcore_axis_name="s"),
    scratch_types=[
        pltpu.VMEM((CHUNK, EB), jnp.int32),
        pltpu.VMEM((CHUNK, EB), jnp.int32),
        pltpu.VMEM((EB, DH), F32),
        pltpu.VMEM((EB, DH), F32),
        pltpu.VMEM((EB, DH), F32),
        pltpu.VMEM((EB, DH), F32),
        pltpu.VMEM((EB, DH), F32),
        pltpu.VMEM_SHARED((NP, DH), F32),
        pltpu.SemaphoreType.DMA,
        pltpu.SemaphoreType.DMA,
        pltpu.SemaphoreType.DMA,
        pltpu.SemaphoreType.DMA,
        pltpu.SemaphoreType.DMA,
        pltpu.SemaphoreType.DMA,
        pltpu.SemaphoreType.DMA,
        pltpu.SemaphoreType.DMA,
        pltpu.SemaphoreType.DMA,
        pltpu.SemaphoreType.DMA,
    ],
)()


def _mm0_body(x_ref, w_ref, mlo_ref, mhi_ref):
    m = jnp.dot(x_ref[...], w_ref[...], preferred_element_type=F32)
    mlo_ref[...] = m[:, :DH]
    mhi_ref[...] = m[:, DH:]


def _mm0(xp, w_top):
    return pl.pallas_call(
        _mm0_body,
        grid=(NP // RB,),
        in_specs=[
            pl.BlockSpec((RB, D_IN), lambda i: (i, 0)),
            pl.BlockSpec((D_IN, DO), lambda i: (0, 0)),
        ],
        out_specs=[
            pl.BlockSpec((RB, DH), lambda i: (i, 0)),
            pl.BlockSpec((RB, DH), lambda i: (i, 0)),
        ],
        out_shape=[jax.ShapeDtypeStruct((NP, DH), F32),
                   jax.ShapeDtypeStruct((NP, DH), F32)],
    )(xp, w_top)


def _gru_core(alo_ref, ahi_ref, h_ref, wil_ref, wih_ref, whh_ref,
              bih_ref, bhh_ref):
    h = h_ref[...]
    gi = (jnp.dot(alo_ref[...], wil_ref[...], preferred_element_type=F32)
          + jnp.dot(ahi_ref[...], wih_ref[...], preferred_element_type=F32)
          + bih_ref[...])
    gh = jnp.dot(h, whh_ref[...], preferred_element_type=F32) + bhh_ref[...]
    r = jax.nn.sigmoid(gi[:, :DO] + gh[:, :DO])
    z = jax.nn.sigmoid(gi[:, DO:2 * DO] + gh[:, DO:2 * DO])
    nn_ = jnp.tanh(gi[:, 2 * DO:] + r * gh[:, 2 * DO:])
    return (1.0 - z) * nn_ + z * h


def _gru_mid_body(alo_ref, ahi_ref, h_ref, wil_ref, wih_ref, whh_ref,
                  bih_ref, bhh_ref, wn_ref, h_out, mlo_ref, mhi_ref):
    hn = _gru_core(alo_ref, ahi_ref, h_ref, wil_ref, wih_ref, whh_ref,
                   bih_ref, bhh_ref)
    h_out[...] = hn
    m = jnp.dot(hn, wn_ref[...], preferred_element_type=F32)
    mlo_ref[...] = m[:, :DH]
    mhi_ref[...] = m[:, DH:]


def _gru_last_body(alo_ref, ahi_ref, h_ref, wil_ref, wih_ref, whh_ref,
                   bih_ref, bhh_ref, h_out):
    h_out[...] = _gru_core(alo_ref, ahi_ref, h_ref, wil_ref, wih_ref,
                           whh_ref, bih_ref, bhh_ref)


_GRU_IN_SPECS = [
    pl.BlockSpec((RB, DH), lambda i: (i, 0)),
    pl.BlockSpec((RB, DH), lambda i: (i, 0)),
    pl.BlockSpec((RB, DO), lambda i: (i, 0)),
    pl.BlockSpec((DH, 3 * DO), lambda i: (0, 0)),
    pl.BlockSpec((DH, 3 * DO), lambda i: (0, 0)),
    pl.BlockSpec((DO, 3 * DO), lambda i: (0, 0)),
    pl.BlockSpec((1, 3 * DO), lambda i: (0, 0)),
    pl.BlockSpec((1, 3 * DO), lambda i: (0, 0)),
]


def _gru_mid(alo, ahi, h, wil, wih, whh, bih, bhh, wn):
    return pl.pallas_call(
        _gru_mid_body,
        grid=(NP // RB,),
        in_specs=_GRU_IN_SPECS + [pl.BlockSpec((DO, DO), lambda i: (0, 0))],
        out_specs=[
            pl.BlockSpec((RB, DO), lambda i: (i, 0)),
            pl.BlockSpec((RB, DH), lambda i: (i, 0)),
            pl.BlockSpec((RB, DH), lambda i: (i, 0)),
        ],
        out_shape=[jax.ShapeDtypeStruct((NP, DO), F32),
                   jax.ShapeDtypeStruct((NP, DH), F32),
                   jax.ShapeDtypeStruct((NP, DH), F32)],
    )(alo, ahi, h, wil, wih, whh, bih, bhh, wn)


def _gru_last(alo, ahi, h, wil, wih, whh, bih, bhh):
    return pl.pallas_call(
        _gru_last_body,
        grid=(NP // RB,),
        in_specs=_GRU_IN_SPECS,
        out_specs=pl.BlockSpec((RB, DO), lambda i: (i, 0)),
        out_shape=jax.ShapeDtypeStruct((NP, DO), F32),
    )(alo, ahi, h, wil, wih, whh, bih, bhh)


def _pool_mlp_body(h_ref, x_ref, b_ref, w1h_ref, w1x_ref, b1_ref,
                   w2_ref, b2_ref, w3_ref, b3_ref, out_ref):
    ids = b_ref[...]
    onehot = (ids == lax.broadcasted_iota(jnp.int32, (GG, NP), 0)).astype(F32)
    counts = jnp.sum(onehot, axis=1, keepdims=True)
    inv = 1.0 / jnp.clip(counts, 1.0, None)
    ph = jnp.dot(onehot, h_ref[...], preferred_element_type=F32) * inv
    px = jnp.dot(onehot, x_ref[...], preferred_element_type=F32) * inv
    h1 = jnp.maximum(
        jnp.dot(ph, w1h_ref[...], preferred_element_type=F32)
        + jnp.dot(px, w1x_ref[...], preferred_element_type=F32)
        + b1_ref[...], 0.0)
    h2 = jnp.maximum(
        jnp.dot(h1, w2_ref[...], preferred_element_type=F32) + b2_ref[...],
        0.0)
    logits = jnp.sum(h2 * w3_ref[...], axis=1, keepdims=True) + b3_ref[...]
    out_ref[...] = jax.nn.sigmoid(logits)


def _pool_mlp(h, xp, batch2, w1h, w1x, b1, w2t, b2, w3r, b3):
    return pl.pallas_call(
        _pool_mlp_body,
        out_shape=jax.ShapeDtypeStruct((GG, 1), F32),
    )(h, xp, batch2, w1h, w1x, b1, w2t, b2, w3r, b3)


def kernel(x, edge_index, batch, W_ggnn, w_ih, w_hh, b_ih, b_hh,
           W1, b1, W2, b2, W3, b3):
    xp = jnp.pad(x, ((0, NP - NN), (0, 0)))
    h = jnp.pad(x, ((0, NP - NN), (0, DO - D_IN)))

    src = edge_index[0]
    dst = edge_index[1]
    padn = EPAD - EE
    src_p = jnp.concatenate([src, jnp.zeros((padn,), jnp.int32)])
    dst_pad = NN + (jnp.arange(padn, dtype=jnp.int32) % (NP - NN))
    dst_p = jnp.concatenate([dst, dst_pad])
    src_t = src_p.reshape(NTILES * NB, EB)
    dst_t = dst_p.reshape(NTILES * NB, EB)
    zrows = jnp.zeros((RPT, DH), F32)

    wil = w_ih.T[:DH]
    wih = w_ih.T[DH:]
    whh = w_hh.T
    bih2 = b_ih.reshape(1, 3 * DO)
    bhh2 = b_hh.reshape(1, 3 * DO)

    batch2 = jnp.pad(batch, (0, NP - NN), constant_values=GG).reshape(1, NP)
    w1h = W1.T[:DO]
    w1x = W1.T[DO:]
    b1r = b1.reshape(1, 256)
    w2t = W2.T
    b2r = b2.reshape(1, 128)
    w3r = W3.reshape(1, 128)
    b3r = b3.reshape(1, 1)

    mlo, mhi = _mm0(xp, W_ggnn[0][:D_IN])

    for i in range(LL):
        alo, ahi = _sc_agg(mlo, mhi, src_t, dst_t, zrows)
        if i == LL - 1:
            h = _gru_last(alo, ahi, h, wil, wih, whh, bih2, bhh2)
        else:
            h, mlo, mhi = _gru_mid(alo, ahi, h, wil, wih, whh, bih2, bhh2,
                                   W_ggnn[i + 1])

    return _pool_mlp(h, xp, batch2, w1h, w1x, b1r, w2t, b2r, w3r, b3r)


# speedup vs baseline: 1.6188x; 1.0304x over previous
import functools

import jax
import jax.numpy as jnp
from jax import lax
from jax.experimental import pallas as pl
from jax.experimental.pallas import tpu as pltpu
from jax.experimental.pallas import tpu_sc as plsc

NN = 10000
NP = 10240
EE = 320000
D_IN = 128
DH = 128
DO = 256
LL = 6
GG = 64

NTILES = 16
EB = 64
NB = 320
CHUNK = 32
NCH = NB // CHUNK
NBUF = 5
EPT = NB * EB
EPAD = EPT * NTILES
RPT = NP // NTILES

RB = 1024
F32 = jnp.float32


def _sc_agg_body(m_lo, m_hi, src_t, dst_t, zrows, agg_lo, agg_hi,
                 src_v, dst_v, buf0, buf1, buf2, buf3, buf4, acc,
                 sem0, sem1, sem2, sem3, sem4,
                 ssem0, ssem1, ssem2, ssem3, ssem4):
    c = lax.axis_index("c")
    s = lax.axis_index("s")
    r0 = s * RPT
    bufs = (buf0, buf1, buf2, buf3, buf4)
    sems = (sem0, sem1, sem2, sem3, sem4)
    ssems = (ssem0, ssem1, ssem2, ssem3, ssem4)

    def run(m_ref, out_ref):
        pltpu.sync_copy(zrows, acc.at[pl.ds(r0, RPT)])
        plsc.subcore_barrier()

        def chunk_step(k, carry):
            base = s * NB + k * CHUNK
            pltpu.sync_copy(src_t.at[pl.ds(base, CHUNK)], src_v)
            pltpu.sync_copy(dst_t.at[pl.ds(base, CHUNK)], dst_v)
            cps = [None] * NBUF
            scps = [None] * NBUF
            for j in range(NBUF - 1):
                cps[j] = pltpu.async_copy(
                    m_ref.at[src_v.at[j]], bufs[j], sems[j])
            for j in range(CHUNK):
                p = j % NBUF
                jn = j + NBUF - 1
                if jn < CHUNK:
                    pn = jn % NBUF
                    if scps[pn] is not None:
                        scps[pn].wait()
                    cps[pn] = pltpu.async_copy(
                        m_ref.at[src_v.at[jn]], bufs[pn], sems[pn])
                cps[p].wait()
                scps[p] = pltpu.async_copy(
                    bufs[p], acc.at[dst_v.at[j]], ssems[p], add=True)
            for p in range(NBUF):
                if scps[p] is not None:
                    scps[p].wait()
            return carry

        lax.fori_loop(0, NCH, chunk_step, 0)
        plsc.subcore_barrier()
        pltpu.sync_copy(acc.at[pl.ds(r0, RPT)], out_ref.at[pl.ds(r0, RPT)])

    @pl.when(c == 0)
    def _():
        run(m_lo, agg_lo)

    @pl.when(c == 1)
    def _():
        run(m_hi, agg_hi)


_sc_agg = functools.partial(
    pl.kernel,
    _sc_agg_body,
    out_type=[jax.ShapeDtypeStruct((NP, DH), F32),
              jax.ShapeDtypeStruct((NP, DH), F32)],
    mesh=plsc.VectorSubcoreMesh(core_axis_name="c", subcore_axis_name="s"),
    scratch_types=[
        pltpu.VMEM((CHUNK, EB), jnp.int32),
        pltpu.VMEM((CHUNK, EB), jnp.int32),
        pltpu.VMEM((EB, DH), F32),
        pltpu.VMEM((EB, DH), F32),
        pltpu.VMEM((EB, DH), F32),
        pltpu.VMEM((EB, DH), F32),
        pltpu.VMEM((EB, DH), F32),
        pltpu.VMEM_SHARED((NP, DH), F32),
        pltpu.SemaphoreType.DMA,
        pltpu.SemaphoreType.DMA,
        pltpu.SemaphoreType.DMA,
        pltpu.SemaphoreType.DMA,
        pltpu.SemaphoreType.DMA,
        pltpu.SemaphoreType.DMA,
        pltpu.SemaphoreType.DMA,
        pltpu.SemaphoreType.DMA,
        pltpu.SemaphoreType.DMA,
        pltpu.SemaphoreType.DMA,
    ],
)()


def _mm0_body(x_ref, w_ref, mlo_ref, mhi_ref):
    m = jnp.dot(x_ref[...], w_ref[...], preferred_element_type=F32)
    mlo_ref[...] = m[:, :DH]
    mhi_ref[...] = m[:, DH:]


def _mm0(xp, w_top):
    return pl.pallas_call(
        _mm0_body,
        grid=(NP // RB,),
        in_specs=[
            pl.BlockSpec((RB, D_IN), lambda i: (i, 0)),
            pl.BlockSpec((D_IN, DO), lambda i: (0, 0)),
        ],
        out_specs=[
            pl.BlockSpec((RB, DH), lambda i: (i, 0)),
            pl.BlockSpec((RB, DH), lambda i: (i, 0)),
        ],
        out_shape=[jax.ShapeDtypeStruct((NP, DH), F32),
                   jax.ShapeDtypeStruct((NP, DH), F32)],
    )(xp, w_top)


def _gru_core(alo_ref, ahi_ref, h_ref, wil_ref, wih_ref, whh_ref,
              bih_ref, bhh_ref):
    h = h_ref[...]
    gi = (jnp.dot(alo_ref[...], wil_ref[...], preferred_element_type=F32)
          + jnp.dot(ahi_ref[...], wih_ref[...], preferred_element_type=F32)
          + bih_ref[...])
    gh = jnp.dot(h, whh_ref[...], preferred_element_type=F32) + bhh_ref[...]
    r = jax.nn.sigmoid(gi[:, :DO] + gh[:, :DO])
    z = jax.nn.sigmoid(gi[:, DO:2 * DO] + gh[:, DO:2 * DO])
    nn_ = jnp.tanh(gi[:, 2 * DO:] + r * gh[:, 2 * DO:])
    return (1.0 - z) * nn_ + z * h


def _gru_mid_body(alo_ref, ahi_ref, h_ref, wil_ref, wih_ref, whh_ref,
                  bih_ref, bhh_ref, wn_ref, h_out, mlo_ref, mhi_ref):
    hn = _gru_core(alo_ref, ahi_ref, h_ref, wil_ref, wih_ref, whh_ref,
                   bih_ref, bhh_ref)
    h_out[...] = hn
    m = jnp.dot(hn, wn_ref[...], preferred_element_type=F32)
    mlo_ref[...] = m[:, :DH]
    mhi_ref[...] = m[:, DH:]


def _gru_last_body(alo_ref, ahi_ref, h_ref, wil_ref, wih_ref, whh_ref,
                   bih_ref, bhh_ref, h_out):
    h_out[...] = _gru_core(alo_ref, ahi_ref, h_ref, wil_ref, wih_ref,
                           whh_ref, bih_ref, bhh_ref)


_GRU_IN_SPECS = [
    pl.BlockSpec((RB, DH), lambda i: (i, 0)),
    pl.BlockSpec((RB, DH), lambda i: (i, 0)),
    pl.BlockSpec((RB, DO), lambda i: (i, 0)),
    pl.BlockSpec((DH, 3 * DO), lambda i: (0, 0)),
    pl.BlockSpec((DH, 3 * DO), lambda i: (0, 0)),
    pl.BlockSpec((DO, 3 * DO), lambda i: (0, 0)),
    pl.BlockSpec((1, 3 * DO), lambda i: (0, 0)),
    pl.BlockSpec((1, 3 * DO), lambda i: (0, 0)),
]


def _gru_mid(alo, ahi, h, wil, wih, whh, bih, bhh, wn):
    return pl.pallas_call(
        _gru_mid_body,
        grid=(NP // RB,),
        in_specs=_GRU_IN_SPECS + [pl.BlockSpec((DO, DO), lambda i: (0, 0))],
        out_specs=[
            pl.BlockSpec((RB, DO), lambda i: (i, 0)),
            pl.BlockSpec((RB, DH), lambda i: (i, 0)),
            pl.BlockSpec((RB, DH), lambda i: (i, 0)),
        ],
        out_shape=[jax.ShapeDtypeStruct((NP, DO), F32),
                   jax.ShapeDtypeStruct((NP, DH), F32),
                   jax.ShapeDtypeStruct((NP, DH), F32)],
    )(alo, ahi, h, wil, wih, whh, bih, bhh, wn)


def _gru_last(alo, ahi, h, wil, wih, whh, bih, bhh):
    return pl.pallas_call(
        _gru_last_body,
        grid=(NP // RB,),
        in_specs=_GRU_IN_SPECS,
        out_specs=pl.BlockSpec((RB, DO), lambda i: (i, 0)),
        out_shape=jax.ShapeDtypeStruct((NP, DO), F32),
    )(alo, ahi, h, wil, wih, whh, bih, bhh)


def _pool_mlp_body(h_ref, x_ref, b_ref, w1h_ref, w1x_ref, b1_ref,
                   w2_ref, b2_ref, w3_ref, b3_ref, out_ref):
    ids = b_ref[...]
    onehot = (ids == lax.broadcasted_iota(jnp.int32, (GG, NP), 0)).astype(F32)
    counts = jnp.sum(onehot, axis=1, keepdims=True)
    inv = 1.0 / jnp.clip(counts, 1.0, None)
    ph = jnp.dot(onehot, h_ref[...], preferred_element_type=F32) * inv
    px = jnp.dot(onehot, x_ref[...], preferred_element_type=F32) * inv
    h1 = jnp.maximum(
        jnp.dot(ph, w1h_ref[...], preferred_element_type=F32)
        + jnp.dot(px, w1x_ref[...], preferred_element_type=F32)
        + b1_ref[...], 0.0)
    h2 = jnp.maximum(
        jnp.dot(h1, w2_ref[...], preferred_element_type=F32) + b2_ref[...],
        0.0)
    logits = jnp.sum(h2 * w3_ref[...], axis=1, keepdims=True) + b3_ref[...]
    out_ref[...] = jax.nn.sigmoid(logits)


def _pool_mlp(h, xp, batch2, w1h, w1x, b1, w2t, b2, w3r, b3):
    return pl.pallas_call(
        _pool_mlp_body,
        out_shape=jax.ShapeDtypeStruct((GG, 1), F32),
    )(h, xp, batch2, w1h, w1x, b1, w2t, b2, w3r, b3)


def kernel(x, edge_index, batch, W_ggnn, w_ih, w_hh, b_ih, b_hh,
           W1, b1, W2, b2, W3, b3):
    xp = jnp.pad(x, ((0, NP - NN), (0, 0)))
    h = jnp.pad(x, ((0, NP - NN), (0, DO - D_IN)))

    src = edge_index[0]
    dst = edge_index[1]
    padn = EPAD - EE
    src_p = jnp.concatenate([src, jnp.zeros((padn,), jnp.int32)])
    dst_pad = NN + (jnp.arange(padn, dtype=jnp.int32) % (NP - NN))
    dst_p = jnp.concatenate([dst, dst_pad])
    src_t = src_p.reshape(NTILES * NB, EB)
    dst_t = dst_p.reshape(NTILES * NB, EB)
    zrows = jnp.zeros((RPT, DH), F32)

    wil = w_ih.T[:DH]
    wih = w_ih.T[DH:]
    whh = w_hh.T
    bih2 = b_ih.reshape(1, 3 * DO)
    bhh2 = b_hh.reshape(1, 3 * DO)

    batch2 = jnp.pad(batch, (0, NP - NN), constant_values=GG).reshape(1, NP)
    w1h = W1.T[:DO]
    w1x = W1.T[DO:]
    b1r = b1.reshape(1, 256)
    w2t = W2.T
    b2r = b2.reshape(1, 128)
    w3r = W3.reshape(1, 128)
    b3r = b3.reshape(1, 1)

    mlo, mhi = _mm0(xp, W_ggnn[0][:D_IN])

    for i in range(LL):
        alo, ahi = _sc_agg(mlo, mhi, src_t, dst_t, zrows)
        if i == LL - 1:
            h = _gru_last(alo, ahi, h, wil, wih, whh, bih2, bhh2)
        else:
            h, mlo, mhi = _gru_mid(alo, ahi, h, wil, wih, whh, bih2, bhh2,
                                   W_ggnn[i + 1])

    return _pool_mlp(h, xp, batch2, w1h, w1x, b1r, w2t, b2r, w3r, b3r)
